# baseline (speedup 1.0000x reference)
import jax
import jax.numpy as jnp
from jax import lax
from jax.experimental import pallas as pl
from jax.experimental.pallas import tpu as pltpu
from jax.experimental.pallas import tpu_sc as plsc

CHUNK = 128
NBUF = 4
LOOKAHEAD = 3
LANES = 16


def _make_body(n_rows, num_segments, ptr_padded_len, nw, ncores):
    rows_per_w = n_rows // nw
    nwin = -(-rows_per_w // CHUNK)
    nouter = -(-nwin // NBUF)
    zrows = ptr_padded_len // LANES
    acc_rows = num_segments + LANES
    zrows_per_sub = acc_rows // (nw // ncores)
    trash = jnp.int32(num_segments)

    nwinp = nouter * NBUF

    def body(x_hbm, ptr_hbm, out_hbm, *scr):
        bufs = scr[:NBUF]
        idxs = scr[NBUF:2 * NBUF]
        zbuf, ptr_full, acc = scr[2 * NBUF:2 * NBUF + 3]
        gsems = scr[2 * NBUF + 3:3 * NBUF + 3]
        ssems = scr[3 * NBUF + 3:4 * NBUF + 3]
        c = lax.axis_index("c")
        s = lax.axis_index("s")
        wid = s * ncores + c
        row_lo = wid * rows_per_w
        row_hi = row_lo + rows_per_w

        pltpu.sync_copy(ptr_hbm, ptr_full)

        def _zrow(r, carry):
            for j in range(8):
                zbuf[r, pl.ds(j * LANES, LANES)] = jnp.zeros((LANES,),
                                                             jnp.float32)
            return carry
        lax.fori_loop(0, zrows_per_sub, _zrow, 0)
        pltpu.sync_copy(zbuf, acc.at[pl.ds(s * zrows_per_sub, zrows_per_sub)])
        plsc.subcore_barrier()

        nbit = max(1, (num_segments).bit_length())

        def _base(k):
            return jnp.minimum(row_lo + k * CHUNK, n_rows - CHUNK)

        def _start_gather(k, b):
            pltpu.async_copy(x_hbm.at[pl.ds(_base(k), CHUNK)],
                             bufs[b], gsems[b])

        for kk in range(LOOKAHEAD):
            _start_gather(kk, kk % NBUF)

        def _window(k, b, it):
            tail = row_lo + k * CHUNK
            base = _base(k)
            buf, idx, gsem, ssem = bufs[b], idxs[b], gsems[b], ssems[b]
            ba = (b + LOOKAHEAD) % NBUF

            @pl.when(k >= NBUF - LOOKAHEAD)
            def _():
                pltpu.make_async_copy(bufs[ba], acc.at[idxs[ba]],
                                      ssems[ba]).wait()

            @pl.when(k + LOOKAHEAD < nwinp)
            def _():
                _start_gather(k + LOOKAHEAD, ba)

            for j in range(8):
                grow = lax.iota(jnp.int32, LANES) + (base + j * LANES)
                blo = jnp.zeros((LANES,), jnp.int32)
                bhi = jnp.full((LANES,), num_segments, jnp.int32)
                for _ in range(nbit):
                    mid = (blo + bhi) // 2
                    v = plsc.load_gather(ptr_full, [mid])
                    le = v <= grow
                    blo = jnp.where(le, mid, blo)
                    bhi = jnp.where(le, bhi, mid)
                valid = (grow >= tail) & (grow < row_hi)
                idx[pl.ds(j * LANES, LANES)] = jnp.where(valid, blo, trash)

            pltpu.make_async_copy(x_hbm.at[pl.ds(base, CHUNK)],
                                  buf, gsem).wait()
            pltpu.async_copy(buf, acc.at[idx], ssem, add=True)

        def _outer(it, carry):
            for b in range(NBUF):
                _window(it * NBUF + b, b, it)
            return carry
        lax.fori_loop(0, nouter, _outer, 0)

        for j in range(nwinp - NBUF + LOOKAHEAD, nwinp):
            b = j % NBUF
            pltpu.make_async_copy(bufs[b], acc.at[idxs[b]], ssems[b]).wait()
        plsc.subcore_barrier()

        out_rows = num_segments // (nw // ncores)
        pltpu.sync_copy(acc.at[pl.ds(s * out_rows, out_rows)],
                        out_hbm.at[c, pl.ds(s * out_rows, out_rows)])

    return body, acc_rows, zrows_per_sub


def _segment_sums(x, ptr_pad, num_segments):
    n_rows, d = x.shape
    info = plsc.get_sparse_core_info()
    ncores, nsub = info.num_cores, info.num_subcores
    nw = ncores * nsub
    assert n_rows % nw == 0
    body, acc_rows, zrows_per_sub = _make_body(
        n_rows, num_segments, ptr_pad.shape[0], nw, ncores)
    mesh = plsc.VectorSubcoreMesh(core_axis_name="c", subcore_axis_name="s")
    return pl.kernel(
        body,
        out_type=jax.ShapeDtypeStruct((ncores, num_segments, d), jnp.float32),
        mesh=mesh,
        compiler_params=pltpu.CompilerParams(use_tc_tiling_on_sc=False,
                                             needs_layout_passes=False),
        scratch_types=(
            [pltpu.VMEM((CHUNK, d), jnp.float32) for _ in range(NBUF)] +
            [pltpu.VMEM((CHUNK,), jnp.int32) for _ in range(NBUF)] +
            [pltpu.VMEM((zrows_per_sub, d), jnp.float32),
             pltpu.VMEM((ptr_pad.shape[0],), jnp.int32),
             pltpu.VMEM_SHARED((acc_rows, d), jnp.float32)]
            + [pltpu.SemaphoreType.DMA for _ in range(2 * NBUF)]
        ),
    )(x, ptr_pad)


def _encode_body(part_ref, scale_ref, mask_ref, w_ref, b_ref, out_ref):
    sums = part_ref[0] + part_ref[1]
    scaled = sums * scale_ref[...]
    out = jnp.dot(scaled, w_ref[...], preferred_element_type=jnp.float32)
    out_ref[...] = jnp.where(mask_ref[...] > 0.0, out + b_ref[...], 0.0)


def _encode(part, scale, mask, W, b2d):
    b_seg = part.shape[1]
    d_out = W.shape[1]
    return pl.pallas_call(
        _encode_body,
        out_shape=jax.ShapeDtypeStruct((b_seg, d_out), jnp.float32),
    )(part, scale, mask, W, b2d)


def kernel(x, ptr, W, b):
    n_rows, _ = x.shape
    num_segments = ptr.shape[0] - 1
    ptr = ptr.astype(jnp.int32)
    pad = -(num_segments + 1) % LANES + LANES
    ptr_pad = jnp.concatenate([ptr, jnp.full((pad,), n_rows, jnp.int32)])
    part = _segment_sums(x, ptr_pad, num_segments)
    counts = ptr[1:] - ptr[:-1]
    scale = (1.0 / jnp.maximum(counts.astype(jnp.float32), 1.0))[:, None]
    mask = (counts > 0).astype(jnp.float32)[:, None]
    return _encode(part, scale, mask, W, b.reshape(1, -1))


# speedup vs baseline: 1.2257x; 1.2257x over previous
import jax
import jax.numpy as jnp
from jax import lax
from jax.experimental import pallas as pl
from jax.experimental.pallas import tpu as pltpu
from jax.experimental.pallas import tpu_sc as plsc

CHUNK = 128
NBUF = 4
LOOKAHEAD = 2
LANES = 16


def _make_body(n_rows, num_segments, ptr_padded_len, nw, ncores):
    rows_per_w = n_rows // nw
    nwin = -(-rows_per_w // CHUNK)
    nouter = -(-nwin // NBUF)
    zrows = ptr_padded_len // LANES
    acc_rows = num_segments + LANES
    zrows_per_sub = acc_rows // (nw // ncores)
    trash = jnp.int32(num_segments)

    nwinp = nouter * NBUF

    def body(x_hbm, ptr_hbm, out_hbm, *scr):
        bufs = scr[:NBUF]
        idxs = scr[NBUF:2 * NBUF]
        zbuf, ptr_full, acc = scr[2 * NBUF:2 * NBUF + 3]
        gsems = scr[2 * NBUF + 3:3 * NBUF + 3]
        ssems = scr[3 * NBUF + 3:4 * NBUF + 3]
        c = lax.axis_index("c")
        s = lax.axis_index("s")
        wid = s * ncores + c
        row_lo = wid * rows_per_w
        row_hi = row_lo + rows_per_w

        pltpu.sync_copy(ptr_hbm, ptr_full)

        def _zrow(r, carry):
            for j in range(8):
                zbuf[r, pl.ds(j * LANES, LANES)] = jnp.zeros((LANES,),
                                                             jnp.float32)
            return carry
        lax.fori_loop(0, zrows_per_sub, _zrow, 0)
        pltpu.sync_copy(zbuf, acc.at[pl.ds(s * zrows_per_sub, zrows_per_sub)])
        plsc.subcore_barrier()

        nbit = max(1, (num_segments).bit_length())

        def _base(k):
            return jnp.minimum(row_lo + k * CHUNK, n_rows - CHUNK)

        def _start_gather(k, b):
            pltpu.async_copy(x_hbm.at[pl.ds(_base(k), CHUNK)],
                             bufs[b], gsems[b])

        for kk in range(LOOKAHEAD):
            _start_gather(kk, kk % NBUF)

        def _window(k, b, it):
            tail = row_lo + k * CHUNK
            base = _base(k)
            buf, idx, gsem, ssem = bufs[b], idxs[b], gsems[b], ssems[b]
            ba = (b + LOOKAHEAD) % NBUF

            @pl.when(k >= NBUF - LOOKAHEAD)
            def _():
                pltpu.make_async_copy(bufs[ba], acc.at[idxs[ba]],
                                      ssems[ba]).wait()

            @pl.when(k + LOOKAHEAD < nwinp)
            def _():
                _start_gather(k + LOOKAHEAD, ba)

            for j in range(8):
                grow = lax.iota(jnp.int32, LANES) + (base + j * LANES)
                blo = jnp.zeros((LANES,), jnp.int32)
                bhi = jnp.full((LANES,), num_segments, jnp.int32)
                for _ in range(nbit):
                    mid = (blo + bhi) // 2
                    v = plsc.load_gather(ptr_full, [mid])
                    le = v <= grow
                    blo = jnp.where(le, mid, blo)
                    bhi = jnp.where(le, bhi, mid)
                valid = (grow >= tail) & (grow < row_hi)
                idx[pl.ds(j * LANES, LANES)] = jnp.where(valid, blo, trash)

            pltpu.make_async_copy(x_hbm.at[pl.ds(base, CHUNK)],
                                  buf, gsem).wait()
            pltpu.async_copy(buf, acc.at[idx], ssem, add=True)

        def _outer(it, carry):
            for b in range(NBUF):
                _window(it * NBUF + b, b, it)
            return carry
        lax.fori_loop(0, nouter, _outer, 0)

        for j in range(nwinp - NBUF + LOOKAHEAD, nwinp):
            b = j % NBUF
            pltpu.make_async_copy(bufs[b], acc.at[idxs[b]], ssems[b]).wait()
        plsc.subcore_barrier()

        out_rows = num_segments // (nw // ncores)
        pltpu.sync_copy(acc.at[pl.ds(s * out_rows, out_rows)],
                        out_hbm.at[c, pl.ds(s * out_rows, out_rows)])

    return body, acc_rows, zrows_per_sub


def _segment_sums(x, ptr_pad, num_segments):
    n_rows, d = x.shape
    info = plsc.get_sparse_core_info()
    ncores, nsub = info.num_cores, info.num_subcores
    nw = ncores * nsub
    assert n_rows % nw == 0
    body, acc_rows, zrows_per_sub = _make_body(
        n_rows, num_segments, ptr_pad.shape[0], nw, ncores)
    mesh = plsc.VectorSubcoreMesh(core_axis_name="c", subcore_axis_name="s")
    return pl.kernel(
        body,
        out_type=jax.ShapeDtypeStruct((ncores, num_segments, d), jnp.float32),
        mesh=mesh,
        compiler_params=pltpu.CompilerParams(use_tc_tiling_on_sc=False,
                                             needs_layout_passes=False),
        scratch_types=(
            [pltpu.VMEM((CHUNK, d), jnp.float32) for _ in range(NBUF)] +
            [pltpu.VMEM((CHUNK,), jnp.int32) for _ in range(NBUF)] +
            [pltpu.VMEM((zrows_per_sub, d), jnp.float32),
             pltpu.VMEM((ptr_pad.shape[0],), jnp.int32),
             pltpu.VMEM_SHARED((acc_rows, d), jnp.float32)]
            + [pltpu.SemaphoreType.DMA for _ in range(2 * NBUF)]
        ),
    )(x, ptr_pad)


def _encode_body(part_ref, scale_ref, mask_ref, w_ref, b_ref, out_ref):
    sums = part_ref[0] + part_ref[1]
    scaled = sums * scale_ref[...]
    out = jnp.dot(scaled, w_ref[...], preferred_element_type=jnp.float32)
    out_ref[...] = jnp.where(mask_ref[...] > 0.0, out + b_ref[...], 0.0)


def _encode(part, scale, mask, W, b2d):
    b_seg = part.shape[1]
    d_out = W.shape[1]
    return pl.pallas_call(
        _encode_body,
        out_shape=jax.ShapeDtypeStruct((b_seg, d_out), jnp.float32),
    )(part, scale, mask, W, b2d)


def kernel(x, ptr, W, b):
    n_rows, _ = x.shape
    num_segments = ptr.shape[0] - 1
    ptr = ptr.astype(jnp.int32)
    pad = -(num_segments + 1) % LANES + LANES
    ptr_pad = jnp.concatenate([ptr, jnp.full((pad,), n_rows, jnp.int32)])
    part = _segment_sums(x, ptr_pad, num_segments)
    counts = ptr[1:] - ptr[:-1]
    scale = (1.0 / jnp.maximum(counts.astype(jnp.float32), 1.0))[:, None]
    mask = (counts > 0).astype(jnp.float32)[:, None]
    return _encode(part, scale, mask, W, b.reshape(1, -1))
